# baseline (speedup 1.0000x reference)
import functools

import jax
import jax.numpy as jnp
from jax import lax
from jax.experimental import pallas as pl
from jax.experimental.pallas import tpu as pltpu
from jax.experimental.pallas import tpu_sc as plsc

B, N = 8, 2048
LOCAL_DIM, POINT_DIM, EF_OUT, K = 64, 3, 128, 5
T = 1024
NT = N // T
DP = 128
HB = B // 2
RH = HB * K * N
NC, NS = 2, 16
NW = NC * NS
RPW = RH // NW
CH = 320
NCH = RPW // CH


def _leaky(x):
    return jnp.where(x >= 0, x, 0.01 * x)


def _knn_body(ptile_ref, pts_ref, lf_ref, idx_ref, tbl_ref):
    b = pl.program_id(0)
    ptsT = jnp.transpose(ptile_ref[0], (1, 0))
    pts = pts_ref[0]
    lf_t = jnp.transpose(lf_ref[0], (1, 0))
    tbl_ref[0] = jnp.concatenate(
        [lf_t, ptsT, jnp.zeros((T, DP - LOCAL_DIM - POINT_DIM), jnp.float32)],
        axis=1)
    inner = lax.dot_general(
        ptsT * -2.0, pts, (((1,), (0,)), ((), ())),
        preferred_element_type=jnp.float32)
    xx_r = jnp.sum(ptsT * ptsT, axis=1, keepdims=True)
    xx_c = jnp.sum(pts * pts, axis=0, keepdims=True)
    d = -xx_r - inner - xx_c
    iota = lax.broadcasted_iota(jnp.int32, (T, N), 1)
    cols = []
    for k in range(K):
        m = jnp.max(d, axis=1, keepdims=True)
        cand = jnp.where(d == m, iota, jnp.int32(N))
        amin = jnp.min(cand, axis=1, keepdims=True)
        cols.append(amin)
        if k < K - 1:
            d = jnp.where(iota == amin, -jnp.inf, d)
    idxs = jnp.concatenate(cols, axis=1) + b * N
    idx_ref[0] = jnp.transpose(idxs, (1, 0))


def _knn_topk(points, local_feats, b0):
    return pl.pallas_call(
        _knn_body,
        grid=(HB, NT),
        in_specs=[
            pl.BlockSpec((1, POINT_DIM, T), lambda b, t: (b0 + b, 0, t)),
            pl.BlockSpec((1, POINT_DIM, N), lambda b, t: (b0 + b, 0, 0)),
            pl.BlockSpec((1, LOCAL_DIM, T), lambda b, t: (b0 + b, 0, t)),
        ],
        out_specs=[
            pl.BlockSpec((1, K, T), lambda b, t: (b, 0, t)),
            pl.BlockSpec((1, T, DP), lambda b, t: (b, t, 0)),
        ],
        out_shape=[
            jax.ShapeDtypeStruct((HB, K, N), jnp.int32),
            jax.ShapeDtypeStruct((HB, N, DP), jnp.float32),
        ],
        compiler_params=pltpu.CompilerParams(
            dimension_semantics=("parallel", "parallel")),
    )(points, points, local_feats)


def _gather_rows(table, idx):
    mesh = plsc.VectorSubcoreMesh(core_axis_name="c", subcore_axis_name="s")

    @functools.partial(
        pl.kernel, mesh=mesh,
        out_type=jax.ShapeDtypeStruct((RH, DP), jnp.float32),
        scratch_types=[
            pltpu.VMEM((RPW,), jnp.int32),
            pltpu.VMEM((CH, DP), jnp.float32),
            pltpu.VMEM((CH, DP), jnp.float32),
            pltpu.SemaphoreType.DMA,
            pltpu.SemaphoreType.DMA,
            pltpu.SemaphoreType.DMA,
            pltpu.SemaphoreType.DMA,
        ],
    )
    def k(table_hbm, idx_hbm, out_hbm, idx_v, rows0, rows1,
          gs0, gs1, os0, os1):
        wid = lax.axis_index("s") * NC + lax.axis_index("c")
        base = wid * RPW
        pltpu.sync_copy(idx_hbm.at[pl.ds(base, RPW)], idx_v)
        rows, gsem, osem = [rows0, rows1], [gs0, gs1], [os0, os1]
        gcp, ocp = [None] * NCH, [None] * NCH
        gcp[0] = pltpu.async_copy(
            table_hbm.at[idx_v.at[pl.ds(0, CH)]], rows[0], gsem[0])
        for j in range(NCH):
            b = j % 2
            if j + 1 < NCH:
                if j >= 1:
                    ocp[j - 1].wait()
                gcp[j + 1] = pltpu.async_copy(
                    table_hbm.at[idx_v.at[pl.ds((j + 1) * CH, CH)]],
                    rows[1 - b], gsem[1 - b])
            gcp[j].wait()
            ocp[j] = pltpu.async_copy(
                rows[b], out_hbm.at[pl.ds(base + j * CH, CH)], osem[b])
        if NCH >= 2:
            ocp[NCH - 2].wait()
        ocp[NCH - 1].wait()

    return k(table, idx)


def _head_body(g_ref, tbl_ref, gf_ref,
               w1f_ref, w1x_ref, b1_ref, w2_ref, b2_ref,
               wm1g_ref, wm1l_ref, wm1f_ref, wm1p_ref, bm1_ref,
               wm2_ref, bm2_ref, wm3_ref, bm3_ref, out_ref, gvec_ref):
    t = pl.program_id(1)

    @pl.when(t == 0)
    def _():
        gvec_ref[...] = (
            jnp.dot(gf_ref[0], wm1g_ref[...],
                    preferred_element_type=jnp.float32) + bm1_ref[...])

    g = g_ref[0].reshape(K * T, DP)
    tbl = tbl_ref[0]
    pts = tbl[:, LOCAL_DIM:LOCAL_DIM + POINT_DIM]
    lf = tbl[:, :LOCAL_DIM]
    lfn = g[:, :LOCAL_DIM]
    ptn = g[:, LOCAL_DIM:LOCAL_DIM + POINT_DIM]
    ctr = jnp.concatenate([ptn[:T], jnp.tile(pts, (K - 1, 1))], axis=0)
    xs = ctr - jnp.concatenate(
        [jnp.zeros((T, POINT_DIM), jnp.float32), ptn[T:]], axis=0)
    h1 = (jnp.dot(lfn, w1f_ref[...], preferred_element_type=jnp.float32)
          + jnp.dot(xs, w1x_ref[...], preferred_element_type=jnp.float32)
          + b1_ref[...])
    h1 = _leaky(h1)
    h2 = jnp.dot(h1, w2_ref[...], preferred_element_type=jnp.float32)
    h2max = jnp.maximum(jnp.maximum(jnp.maximum(h2[:T], h2[T:2 * T]),
                                    jnp.maximum(h2[2 * T:3 * T],
                                                h2[3 * T:4 * T])),
                        h2[4 * T:])
    fmax = h2max + b2_ref[...]
    o1 = (gvec_ref[...]
          + jnp.dot(lf, wm1l_ref[...], preferred_element_type=jnp.float32)
          + jnp.dot(fmax, wm1f_ref[...], preferred_element_type=jnp.float32)
          + jnp.dot(pts, wm1p_ref[...], preferred_element_type=jnp.float32))
    o1 = _leaky(o1)
    o2 = _leaky(jnp.dot(o1, wm2_ref[...], preferred_element_type=jnp.float32)
                + bm2_ref[...])
    o3 = (jnp.dot(o2, wm3_ref[...], preferred_element_type=jnp.float32)
          + bm3_ref[...])
    out_ref[0] = jnp.transpose(o3, (1, 0))


def _head(gathered, tbl, gf, w1f, w1x, b1, w2, b2,
          wm1g, wm1l, wm1f, wm1p, bm1, wm2, bm2, wm3, bm3, b0):
    full = lambda shape: pl.BlockSpec(shape, lambda b, t: tuple(0 for _ in shape))
    return pl.pallas_call(
        _head_body,
        grid=(HB, NT),
        in_specs=[
            pl.BlockSpec((1, K, T, DP), lambda b, t: (b, 0, t, 0)),
            pl.BlockSpec((1, T, DP), lambda b, t: (b, t, 0)),
            pl.BlockSpec((1, 1, 1024), lambda b, t: (b0 + b, 0, 0)),
            full((LOCAL_DIM, 256)), full((POINT_DIM, 256)), full((1, 256)),
            full((256, EF_OUT)), full((1, EF_OUT)),
            full((1024, 1024)), full((LOCAL_DIM, 1024)), full((EF_OUT, 1024)),
            full((POINT_DIM, 1024)), full((1, 1024)),
            full((1024, 256)), full((1, 256)), full((256, 1)), full((1, 1)),
        ],
        out_specs=pl.BlockSpec((1, 1, T), lambda b, t: (b, 0, t)),
        out_shape=jax.ShapeDtypeStruct((HB, 1, N), jnp.float32),
        scratch_shapes=[pltpu.VMEM((1, 1024), jnp.float32)],
        compiler_params=pltpu.CompilerParams(
            dimension_semantics=("parallel", "arbitrary")),
    )(gathered, tbl, gf, w1f, w1x, b1, w2, b2,
      wm1g, wm1l, wm1f, wm1p, bm1, wm2, bm2, wm3, bm3)


def kernel(global_feat, points, local_feats, W1, b1, W2, b2,
           Wm1, bm1, Wm2, bm2, Wm3, bm3):
    W1T = W1.T
    Wm1T = Wm1.T
    gf3 = global_feat[:, None, :]
    outs = []
    for b0 in (0, HB):
        idx, table = _knn_topk(points, local_feats, b0)
        gathered = _gather_rows(table.reshape(HB * N, DP),
                                idx.reshape(RH)).reshape(HB, K, N, DP)
        outs.append(_head(
            gathered, table, gf3,
            W1T[:LOCAL_DIM], W1T[LOCAL_DIM:], b1.reshape(1, 256),
            W2.T, b2.reshape(1, EF_OUT),
            Wm1T[:1024], Wm1T[1024:1024 + LOCAL_DIM],
            Wm1T[1024 + LOCAL_DIM:1024 + LOCAL_DIM + EF_OUT],
            Wm1T[1024 + LOCAL_DIM + EF_OUT:], bm1.reshape(1, 1024),
            Wm2.T, bm2.reshape(1, 256), Wm3.T, bm3.reshape(1, 1), b0))
    return jnp.concatenate(outs, axis=0)


# speedup vs baseline: 1.1032x; 1.1032x over previous
import functools

import jax
import jax.numpy as jnp
from jax import lax
from jax.experimental import pallas as pl
from jax.experimental.pallas import tpu as pltpu
from jax.experimental.pallas import tpu_sc as plsc

B, N = 8, 2048
LOCAL_DIM, POINT_DIM, EF_OUT, K = 64, 3, 128, 5
T = 1024
NT = N // T
DP = 128
HB = B // 2
RH = HB * K * N
NC, NS = 2, 16
NW = NC * NS
RPW = RH // NW
CH = 320
NCH = RPW // CH


def _leaky(x):
    return jnp.where(x >= 0, x, 0.01 * x)


def _knn_body(ptile_ref, pts_ref, lf_ref, idx_ref, tbl_ref):
    b = pl.program_id(0)
    ptsT = jnp.transpose(ptile_ref[0], (1, 0))
    pts = pts_ref[0]
    lf_t = jnp.transpose(lf_ref[0], (1, 0))
    tbl_ref[0] = jnp.concatenate(
        [lf_t, ptsT, jnp.zeros((T, DP - LOCAL_DIM - POINT_DIM), jnp.float32)],
        axis=1)
    inner = lax.dot_general(
        ptsT * -2.0, pts, (((1,), (0,)), ((), ())),
        preferred_element_type=jnp.float32)
    xx_r = jnp.sum(ptsT * ptsT, axis=1, keepdims=True)
    xx_c = jnp.sum(pts * pts, axis=0, keepdims=True)
    d = -xx_r - inner - xx_c
    iota = lax.broadcasted_iota(jnp.int32, (T, N), 1).astype(jnp.float32)
    cols = []
    for k in range(K):
        m = jnp.max(d, axis=1, keepdims=True)
        cand = jnp.where(d == m, iota, jnp.float32(N))
        amin = jnp.min(cand, axis=1, keepdims=True)
        cols.append(amin)
        if k < K - 1:
            d = jnp.where(cand == amin, -jnp.inf, d)
    idxs = (jnp.concatenate(cols, axis=1).astype(jnp.int32)
            + b * N)
    idx_ref[0] = jnp.transpose(idxs, (1, 0))


def _knn_topk(points, local_feats, b0):
    return pl.pallas_call(
        _knn_body,
        grid=(HB, NT),
        in_specs=[
            pl.BlockSpec((1, POINT_DIM, T), lambda b, t: (b0 + b, 0, t)),
            pl.BlockSpec((1, POINT_DIM, N), lambda b, t: (b0 + b, 0, 0)),
            pl.BlockSpec((1, LOCAL_DIM, T), lambda b, t: (b0 + b, 0, t)),
        ],
        out_specs=[
            pl.BlockSpec((1, K, T), lambda b, t: (b, 0, t)),
            pl.BlockSpec((1, T, DP), lambda b, t: (b, t, 0)),
        ],
        out_shape=[
            jax.ShapeDtypeStruct((HB, K, N), jnp.int32),
            jax.ShapeDtypeStruct((HB, N, DP), jnp.float32),
        ],
        compiler_params=pltpu.CompilerParams(
            dimension_semantics=("parallel", "parallel")),
    )(points, points, local_feats)


def _gather_rows(table, idx):
    mesh = plsc.VectorSubcoreMesh(core_axis_name="c", subcore_axis_name="s")

    @functools.partial(
        pl.kernel, mesh=mesh,
        out_type=jax.ShapeDtypeStruct((RH, DP), jnp.float32),
        scratch_types=[
            pltpu.VMEM((RPW,), jnp.int32),
            pltpu.VMEM((CH, DP), jnp.float32),
            pltpu.VMEM((CH, DP), jnp.float32),
            pltpu.SemaphoreType.DMA,
            pltpu.SemaphoreType.DMA,
            pltpu.SemaphoreType.DMA,
            pltpu.SemaphoreType.DMA,
        ],
    )
    def k(table_hbm, idx_hbm, out_hbm, idx_v, rows0, rows1,
          gs0, gs1, os0, os1):
        wid = lax.axis_index("s") * NC + lax.axis_index("c")
        base = wid * RPW
        pltpu.sync_copy(idx_hbm.at[pl.ds(base, RPW)], idx_v)
        rows, gsem, osem = [rows0, rows1], [gs0, gs1], [os0, os1]
        gcp, ocp = [None] * NCH, [None] * NCH
        gcp[0] = pltpu.async_copy(
            table_hbm.at[idx_v.at[pl.ds(0, CH)]], rows[0], gsem[0])
        for j in range(NCH):
            b = j % 2
            if j + 1 < NCH:
                if j >= 1:
                    ocp[j - 1].wait()
                gcp[j + 1] = pltpu.async_copy(
                    table_hbm.at[idx_v.at[pl.ds((j + 1) * CH, CH)]],
                    rows[1 - b], gsem[1 - b])
            gcp[j].wait()
            ocp[j] = pltpu.async_copy(
                rows[b], out_hbm.at[pl.ds(base + j * CH, CH)], osem[b])
        if NCH >= 2:
            ocp[NCH - 2].wait()
        ocp[NCH - 1].wait()

    return k(table, idx)


def _head_body(g_ref, tbl_ref, gf_ref,
               w1f_ref, w1x_ref, b1_ref, w2_ref, b2_ref,
               wm1g_ref, wm1l_ref, wm1f_ref, wm1p_ref, bm1_ref,
               wm2_ref, bm2_ref, wm3_ref, bm3_ref, out_ref, gvec_ref):
    t = pl.program_id(1)

    @pl.when(t == 0)
    def _():
        gvec_ref[...] = (
            jnp.dot(gf_ref[0], wm1g_ref[...],
                    preferred_element_type=jnp.float32) + bm1_ref[...])

    g = g_ref[0].reshape(K * T, DP)
    tbl = tbl_ref[0]
    pts = tbl[:, LOCAL_DIM:LOCAL_DIM + POINT_DIM]
    lf = tbl[:, :LOCAL_DIM]
    lfn = g[:, :LOCAL_DIM]
    ptn = g[:, LOCAL_DIM:LOCAL_DIM + POINT_DIM]
    ctr = jnp.concatenate([ptn[:T], jnp.tile(pts, (K - 1, 1))], axis=0)
    xs = ctr - jnp.concatenate(
        [jnp.zeros((T, POINT_DIM), jnp.float32), ptn[T:]], axis=0)
    h1 = (jnp.dot(lfn, w1f_ref[...], preferred_element_type=jnp.float32)
          + jnp.dot(xs, w1x_ref[...], preferred_element_type=jnp.float32)
          + b1_ref[...])
    h1 = _leaky(h1)
    h2 = jnp.dot(h1, w2_ref[...], preferred_element_type=jnp.float32)
    h2max = jnp.maximum(jnp.maximum(jnp.maximum(h2[:T], h2[T:2 * T]),
                                    jnp.maximum(h2[2 * T:3 * T],
                                                h2[3 * T:4 * T])),
                        h2[4 * T:])
    fmax = h2max + b2_ref[...]
    o1 = (gvec_ref[...]
          + jnp.dot(lf, wm1l_ref[...], preferred_element_type=jnp.float32)
          + jnp.dot(fmax, wm1f_ref[...], preferred_element_type=jnp.float32)
          + jnp.dot(pts, wm1p_ref[...], preferred_element_type=jnp.float32))
    o1 = _leaky(o1)
    o2 = _leaky(jnp.dot(o1, wm2_ref[...], preferred_element_type=jnp.float32)
                + bm2_ref[...])
    o3 = (jnp.dot(o2, wm3_ref[...], preferred_element_type=jnp.float32)
          + bm3_ref[...])
    out_ref[0] = jnp.transpose(o3, (1, 0))


def _head(gathered, tbl, gf, w1f, w1x, b1, w2, b2,
          wm1g, wm1l, wm1f, wm1p, bm1, wm2, bm2, wm3, bm3, b0):
    full = lambda shape: pl.BlockSpec(shape, lambda b, t: tuple(0 for _ in shape))
    return pl.pallas_call(
        _head_body,
        grid=(HB, NT),
        in_specs=[
            pl.BlockSpec((1, K, T, DP), lambda b, t: (b, 0, t, 0)),
            pl.BlockSpec((1, T, DP), lambda b, t: (b, t, 0)),
            pl.BlockSpec((1, 1, 1024), lambda b, t: (b0 + b, 0, 0)),
            full((LOCAL_DIM, 256)), full((POINT_DIM, 256)), full((1, 256)),
            full((256, EF_OUT)), full((1, EF_OUT)),
            full((1024, 1024)), full((LOCAL_DIM, 1024)), full((EF_OUT, 1024)),
            full((POINT_DIM, 1024)), full((1, 1024)),
            full((1024, 256)), full((1, 256)), full((256, 1)), full((1, 1)),
        ],
        out_specs=pl.BlockSpec((1, 1, T), lambda b, t: (b, 0, t)),
        out_shape=jax.ShapeDtypeStruct((HB, 1, N), jnp.float32),
        scratch_shapes=[pltpu.VMEM((1, 1024), jnp.float32)],
        compiler_params=pltpu.CompilerParams(
            dimension_semantics=("parallel", "arbitrary")),
    )(gathered, tbl, gf, w1f, w1x, b1, w2, b2,
      wm1g, wm1l, wm1f, wm1p, bm1, wm2, bm2, wm3, bm3)


def kernel(global_feat, points, local_feats, W1, b1, W2, b2,
           Wm1, bm1, Wm2, bm2, Wm3, bm3):
    W1T = W1.T
    Wm1T = Wm1.T
    gf3 = global_feat[:, None, :]
    outs = []
    for b0 in (0, HB):
        idx, table = _knn_topk(points, local_feats, b0)
        gathered = _gather_rows(table.reshape(HB * N, DP),
                                idx.reshape(RH)).reshape(HB, K, N, DP)
        outs.append(_head(
            gathered, table, gf3,
            W1T[:LOCAL_DIM], W1T[LOCAL_DIM:], b1.reshape(1, 256),
            W2.T, b2.reshape(1, EF_OUT),
            Wm1T[:1024], Wm1T[1024:1024 + LOCAL_DIM],
            Wm1T[1024 + LOCAL_DIM:1024 + LOCAL_DIM + EF_OUT],
            Wm1T[1024 + LOCAL_DIM + EF_OUT:], bm1.reshape(1, 1024),
            Wm2.T, bm2.reshape(1, 256), Wm3.T, bm3.reshape(1, 1), b0))
    return jnp.concatenate(outs, axis=0)
